# baseline (speedup 1.0000x reference)
import functools

import jax
import jax.numpy as jnp
from jax import lax
from jax.experimental import pallas as pl
from jax.experimental.pallas import tpu as pltpu
from jax.experimental.pallas import tpu_sc as plsc

B, L, D = 4096, 200, 64
N = B * L
NC, NS = 2, 16
NW = NC * NS
W = N // NW
C = 512
KJ = C // 128
G = W // C
NT = 6

IDX_B = NT * KJ * 128 * 4
T0_B = KJ * 128 * D * 4
ADD_B = (NT - 1) * T0_B
OUT_B = C * D * 4


def _sc_embed_sum(idxs, tables):
    mesh = plsc.VectorSubcoreMesh(core_axis_name="c", subcore_axis_name="s")

    @functools.partial(
        pl.kernel,
        out_type=jax.ShapeDtypeStruct((N, D), jnp.float32),
        mesh=mesh,
        scratch_types=[
            pltpu.VMEM((2, NT, KJ, 128), jnp.int32),
            pltpu.VMEM((2, C, D), jnp.float32),
            pltpu.SemaphoreType.DMA((2,)),
            pltpu.SemaphoreType.DMA((2,)),
            pltpu.SemaphoreType.DMA((2,)),
            pltpu.SemaphoreType.DMA((2,)),
        ],
        compiler_params=pltpu.CompilerParams(use_tc_tiling_on_sc=False),
    )
    def k(i0, i1, i2, i3, i4, i5, t0, t1, t2, t3, t4, t5, out_hbm,
          idx_v, acc, s_idx, s_t0, s_add, s_out):
        idx_hbm = [i0, i1, i2, i3, i4, i5]
        tabs = [t0, t1, t2, t3, t4, t5]
        wid = lax.axis_index("s") * NC + lax.axis_index("c")

        def issue_idx(g, p):
            j0 = pl.multiple_of((wid * W + g * C) // 128, KJ)
            for t in range(NT):
                pltpu.async_copy(idx_hbm[t].at[pl.ds(j0, KJ), :],
                                 idx_v.at[p, t], s_idx.at[p])

        def issue_t0(p):
            for j in range(KJ):
                pltpu.async_copy(tabs[0].at[idx_v.at[p, 0, j]],
                                 acc.at[p, pl.ds(j * 128, 128)], s_t0.at[p])

        def issue_adds(p):
            for t in range(1, NT):
                for j in range(KJ):
                    pltpu.async_copy(tabs[t].at[idx_v.at[p, t, j]],
                                     acc.at[p, pl.ds(j * 128, 128)],
                                     s_add.at[p], add=True)

        def issue_out(g, p):
            row0 = wid * W + g * C
            pltpu.async_copy(acc.at[p], out_hbm.at[pl.ds(row0, C)],
                             s_out.at[p])

        def wait_idx(p):
            for t in range(NT):
                pltpu.make_async_copy(idx_hbm[t].at[pl.ds(0, KJ), :],
                                      idx_v.at[p, t], s_idx.at[p]).wait()

        def wait_t0(p):
            pltpu.make_async_copy(out_hbm.at[pl.ds(0, C)], acc.at[p],
                                  s_t0.at[p]).wait()

        def wait_adds(p):
            for _ in range(NT - 1):
                pltpu.make_async_copy(out_hbm.at[pl.ds(0, C)], acc.at[p],
                                      s_add.at[p]).wait()

        def wait_out(p):
            pltpu.make_async_copy(acc.at[p], out_hbm.at[pl.ds(0, C)],
                                  s_out.at[p]).wait()

        def pair(i, carry):
            a, b = 2 * i, 2 * i + 1
            wait_t0(0)
            issue_adds(0)
            wait_idx(1)

            @pl.when(i > 0)
            def _():
                wait_out(1)

            issue_t0(1)
            wait_adds(0)
            issue_out(a, 0)

            @pl.when(a + 2 < G)
            def _():
                issue_idx(a + 2, 0)

            wait_t0(1)
            issue_adds(1)

            @pl.when(a + 2 < G)
            def _():
                wait_idx(0)
                wait_out(0)
                issue_t0(0)

            wait_adds(1)
            issue_out(b, 1)

            @pl.when(b + 2 < G)
            def _():
                issue_idx(b + 2, 1)

            return carry

        issue_idx(0, 0)
        wait_idx(0)
        issue_t0(0)
        issue_idx(1, 1)
        lax.fori_loop(0, G // 2, pair, 0)
        wait_out(0)
        wait_out(1)

    return k(*idxs, *tables)


R = 2048


def _tc_layernorm(hm2, esum, gamma, beta):
    def body(hm_ref, e_ref, g_ref, b_ref, o_ref):
        x = hm_ref[...] + e_ref[...]
        mu = jnp.mean(x, axis=1, keepdims=True)
        xc = x - mu
        var = jnp.mean(xc * xc, axis=1, keepdims=True)
        inv = lax.rsqrt(var + 1e-5)
        o_ref[...] = xc * inv * g_ref[...] + b_ref[...]

    return pl.pallas_call(
        body,
        grid=(N // R,),
        in_specs=[
            pl.BlockSpec((R, D), lambda i: (i, 0)),
            pl.BlockSpec((R, D), lambda i: (i, 0)),
            pl.BlockSpec((1, D), lambda i: (0, 0)),
            pl.BlockSpec((1, D), lambda i: (0, 0)),
        ],
        out_specs=pl.BlockSpec((R, D), lambda i: (i, 0)),
        out_shape=jax.ShapeDtypeStruct((N, D), jnp.float32),
    )(hm2, esum, gamma, beta)


def kernel(target, hm, dhi, dni, ws, rh, t, r_table, dhi_table, dni_table,
           ws_table, rh_table, t_table, gamma, beta):
    idxs = [a.astype(jnp.int32).reshape(N // 128, 128)
            for a in (target, dhi, dni, ws, rh, t)]
    tables = [r_table, dhi_table, dni_table, ws_table, rh_table, t_table]
    esum = _sc_embed_sum(idxs, tables)
    out = _tc_layernorm(
        hm.reshape(N, D), esum,
        gamma.reshape(1, D), beta.reshape(1, D))
    return out.reshape(B, L, D)


# speedup vs baseline: 2.2987x; 2.2987x over previous
import functools

import jax
import jax.numpy as jnp
from jax import lax
from jax.experimental import pallas as pl
from jax.experimental.pallas import tpu as pltpu
from jax.experimental.pallas import tpu_sc as plsc

B, L, D = 4096, 200, 64
N = B * L
NC, NS = 2, 16
NW = NC * NS
W = N // NW
C = 512
KJ = C // 128
G = W // C
NT = 6
VS = [101, 55, 37, 24, 95, 13]

IDX_B = NT * KJ * 128 * 4
T0_B = KJ * 128 * D * 4
ADD_B = (NT - 1) * T0_B
OUT_B = C * D * 4


def _sc_embed_sum(idxs, tables):
    mesh = plsc.VectorSubcoreMesh(core_axis_name="c", subcore_axis_name="s")

    @functools.partial(
        pl.kernel,
        out_type=jax.ShapeDtypeStruct((N, D), jnp.float32),
        mesh=mesh,
        scratch_types=[
            pltpu.VMEM((2, NT, KJ, 128), jnp.int32),
            pltpu.VMEM((2, C, D), jnp.float32),
            pltpu.SemaphoreType.DMA((2,)),
            pltpu.SemaphoreType.DMA((2,)),
            pltpu.SemaphoreType.DMA((2,)),
            pltpu.SemaphoreType.DMA((2,)),
        ],
        compiler_params=pltpu.CompilerParams(use_tc_tiling_on_sc=False),
    )
    def k(i0, i1, i2, i3, i4, i5, t0, t1, t2, t3, t4, t5, out_hbm,
          idx_v, acc, s_idx, s_t0, s_add, s_out):
        idx_hbm = [i0, i1, i2, i3, i4, i5]
        tabs = [t0, t1, t2, t3, t4, t5]
        wid = lax.axis_index("s") * NC + lax.axis_index("c")

        def adjust_idx(p):
            for t in range(NT):
                off = wid * VS[t]
                for j in range(KJ):
                    for k in range(128 // 16):
                        sl = (p, t, j, pl.ds(k * 16, 16))
                        idx_v[sl] = idx_v[sl] + off

        def issue_idx(g, p):
            j0 = pl.multiple_of((wid * W + g * C) // 128, KJ)
            for t in range(NT):
                pltpu.async_copy(idx_hbm[t].at[pl.ds(j0, KJ), :],
                                 idx_v.at[p, t], s_idx.at[p])

        def issue_t0(p):
            for j in range(KJ):
                pltpu.async_copy(tabs[0].at[idx_v.at[p, 0, j]],
                                 acc.at[p, pl.ds(j * 128, 128)], s_t0.at[p])

        def issue_adds(p):
            for t in range(1, NT):
                for j in range(KJ):
                    pltpu.async_copy(tabs[t].at[idx_v.at[p, t, j]],
                                     acc.at[p, pl.ds(j * 128, 128)],
                                     s_add.at[p], add=True)

        def issue_out(g, p):
            row0 = wid * W + g * C
            pltpu.async_copy(acc.at[p], out_hbm.at[pl.ds(row0, C)],
                             s_out.at[p])

        def wait_idx(p):
            for t in range(NT):
                pltpu.make_async_copy(idx_hbm[t].at[pl.ds(0, KJ), :],
                                      idx_v.at[p, t], s_idx.at[p]).wait()

        def wait_t0(p):
            pltpu.make_async_copy(out_hbm.at[pl.ds(0, C)], acc.at[p],
                                  s_t0.at[p]).wait()

        def wait_adds(p):
            for _ in range(NT - 1):
                pltpu.make_async_copy(out_hbm.at[pl.ds(0, C)], acc.at[p],
                                      s_add.at[p]).wait()

        def wait_out(p):
            pltpu.make_async_copy(acc.at[p], out_hbm.at[pl.ds(0, C)],
                                  s_out.at[p]).wait()

        def pair(i, carry):
            a, b = 2 * i, 2 * i + 1
            wait_t0(0)
            issue_adds(0)
            wait_idx(1)
            adjust_idx(1)

            @pl.when(i > 0)
            def _():
                wait_out(1)

            issue_t0(1)
            wait_adds(0)
            issue_out(a, 0)

            @pl.when(a + 2 < G)
            def _():
                issue_idx(a + 2, 0)

            wait_t0(1)
            issue_adds(1)

            @pl.when(a + 2 < G)
            def _():
                wait_idx(0)
                adjust_idx(0)
                wait_out(0)
                issue_t0(0)

            wait_adds(1)
            issue_out(b, 1)

            @pl.when(b + 2 < G)
            def _():
                issue_idx(b + 2, 1)

            return carry

        issue_idx(0, 0)
        wait_idx(0)
        adjust_idx(0)
        issue_t0(0)
        issue_idx(1, 1)
        lax.fori_loop(0, G // 2, pair, 0)
        wait_out(0)
        wait_out(1)

    return k(*idxs, *tables)


R = 2048


def _tc_layernorm(hm2, esum, gamma, beta):
    def body(hm_ref, e_ref, g_ref, b_ref, o_ref):
        x = hm_ref[...] + e_ref[...]
        mu = jnp.mean(x, axis=1, keepdims=True)
        xc = x - mu
        var = jnp.mean(xc * xc, axis=1, keepdims=True)
        inv = lax.rsqrt(var + 1e-5)
        o_ref[...] = xc * inv * g_ref[...] + b_ref[...]

    return pl.pallas_call(
        body,
        grid=(N // R,),
        in_specs=[
            pl.BlockSpec((R, D), lambda i: (i, 0)),
            pl.BlockSpec((R, D), lambda i: (i, 0)),
            pl.BlockSpec((1, D), lambda i: (0, 0)),
            pl.BlockSpec((1, D), lambda i: (0, 0)),
        ],
        out_specs=pl.BlockSpec((R, D), lambda i: (i, 0)),
        out_shape=jax.ShapeDtypeStruct((N, D), jnp.float32),
    )(hm2, esum, gamma, beta)


def kernel(target, hm, dhi, dni, ws, rh, t, r_table, dhi_table, dni_table,
           ws_table, rh_table, t_table, gamma, beta):
    idxs = [a.astype(jnp.int32).reshape(N // 128, 128)
            for a in (target, dhi, dni, ws, rh, t)]
    tables = [jnp.tile(tb, (NW, 1))
              for tb in (r_table, dhi_table, dni_table, ws_table, rh_table,
                         t_table)]
    esum = _sc_embed_sum(idxs, tables)
    out = _tc_layernorm(
        hm.reshape(N, D), esum,
        gamma.reshape(1, D), beta.reshape(1, D))
    return out.reshape(B, L, D)


# speedup vs baseline: 2.5258x; 1.0988x over previous
import functools

import jax
import jax.numpy as jnp
from jax import lax
from jax.experimental import pallas as pl
from jax.experimental.pallas import tpu as pltpu
from jax.experimental.pallas import tpu_sc as plsc

B, L, D = 4096, 200, 64
N = B * L
NC, NS = 2, 16
NW = NC * NS
W = N // NW
C = 512
KJ = C // 128
P = W // (2 * C)
NT = 3
PVS = [1320, 2280, 2040]

IDXP_B = NT * 2 * KJ * 128 * 4
T0_B = KJ * 128 * D * 4
ADD_B = (NT - 1) * T0_B
OUT_B = C * D * 4


def _tc_pack_indices(target, dhi, dni, ws, rh, t):
    def body(tg_ref, dh_ref, dn_ref, ws_ref, rh_ref, tt_ref, o_ref):
        w = pl.program_id(0)
        o_ref[0] = tg_ref[...] * 13 + tt_ref[...] + w * PVS[0]
        o_ref[1] = rh_ref[...] * 24 + ws_ref[...] + w * PVS[1]
        o_ref[2] = dh_ref[...] * 37 + dn_ref[...] + w * PVS[2]

    bs = pl.BlockSpec((B // NW, L), lambda i: (i, 0))
    return pl.pallas_call(
        body,
        grid=(NW,),
        in_specs=[bs] * 6,
        out_specs=pl.BlockSpec((NT, B // NW, L), lambda i: (0, i, 0)),
        out_shape=jax.ShapeDtypeStruct((NT, B, L), jnp.int32),
    )(target, dhi, dni, ws, rh, t)


def _one_hot_rows(n_rows, va, vb):
    r = lax.broadcasted_iota(jnp.int32, (n_rows, 1), 0)
    oa = (r // vb == lax.broadcasted_iota(jnp.int32, (n_rows, va), 1))
    ob = (r % vb == lax.broadcasted_iota(jnp.int32, (n_rows, vb), 1))
    return oa.astype(jnp.float32), ob.astype(jnp.float32)


def _tc_build_tables(r_table, t_table, rh_table, ws_table, dhi_table,
                     dni_table):
    def body(ra_ref, tt_ref, rh_ref, ws_ref, dh_ref, dn_ref,
             oa_ref, ob_ref, oc_ref):
        def pair(a_ref, b_ref, n_rows, va, vb):
            oa, ob = _one_hot_rows(n_rows, va, vb)
            hi = lax.Precision.HIGHEST
            return (jnp.dot(oa, a_ref[...], precision=hi,
                            preferred_element_type=jnp.float32)
                    + jnp.dot(ob, b_ref[...], precision=hi,
                              preferred_element_type=jnp.float32))

        oa_ref[...] = pair(ra_ref, tt_ref, PVS[0], 101, 13)
        ob_ref[...] = pair(rh_ref, ws_ref, PVS[1], 95, 24)
        oc_ref[...] = pair(dh_ref, dn_ref, PVS[2], 55, 37)

    full = lambda s: pl.BlockSpec(s, lambda i: (0, 0))
    return pl.pallas_call(
        body,
        grid=(NW,),
        in_specs=[full((101, D)), full((13, D)), full((95, D)),
                  full((24, D)), full((55, D)), full((37, D))],
        out_specs=[pl.BlockSpec((PVS[0], D), lambda i: (i, 0)),
                   pl.BlockSpec((PVS[1], D), lambda i: (i, 0)),
                   pl.BlockSpec((PVS[2], D), lambda i: (i, 0))],
        out_shape=[jax.ShapeDtypeStruct((NW * PVS[0], D), jnp.float32),
                   jax.ShapeDtypeStruct((NW * PVS[1], D), jnp.float32),
                   jax.ShapeDtypeStruct((NW * PVS[2], D), jnp.float32)],
    )(r_table, t_table, rh_table, ws_table, dhi_table, dni_table)


def _sc_embed_sum(idx_packed, tables):
    mesh = plsc.VectorSubcoreMesh(core_axis_name="c", subcore_axis_name="s")

    @functools.partial(
        pl.kernel,
        out_type=jax.ShapeDtypeStruct((N, D), jnp.float32),
        mesh=mesh,
        scratch_types=[
            pltpu.VMEM((2, NT, 2 * KJ, 128), jnp.int32),
            pltpu.VMEM((2, C, D), jnp.float32),
            pltpu.SemaphoreType.DMA,
            pltpu.SemaphoreType.DMA((2,)),
            pltpu.SemaphoreType.DMA((2,)),
            pltpu.SemaphoreType.DMA((2,)),
        ],
        compiler_params=pltpu.CompilerParams(use_tc_tiling_on_sc=False),
    )
    def k(idx_hbm, t0, t1, t2, out_hbm, idx_v, acc, s_idx, s_t0, s_add,
          s_out):
        tabs = [t0, t1, t2]
        wid = lax.axis_index("s") * NC + lax.axis_index("c")

        def fetch_idx(pair_i, slot):
            r0 = pl.multiple_of(wid * (W // 128) + pair_i * 2 * KJ, 8)
            pltpu.async_copy(idx_hbm.at[:, pl.ds(r0, 2 * KJ), :],
                             idx_v.at[slot], s_idx)

        def wait_idx():
            pltpu.make_async_copy(idx_hbm.at[:, pl.ds(0, 2 * KJ), :],
                                  idx_v.at[0], s_idx).wait()

        def issue_t0(slot, h):
            for j in range(KJ):
                pltpu.async_copy(tabs[0].at[idx_v.at[slot, 0, h * KJ + j]],
                                 acc.at[h, pl.ds(j * 128, 128)], s_t0.at[h])

        def issue_adds(slot, h):
            for t in range(1, NT):
                for j in range(KJ):
                    pltpu.async_copy(
                        tabs[t].at[idx_v.at[slot, t, h * KJ + j]],
                        acc.at[h, pl.ds(j * 128, 128)], s_add.at[h],
                        add=True)

        def issue_out(g, h):
            row0 = wid * W + g * C
            pltpu.async_copy(acc.at[h], out_hbm.at[pl.ds(row0, C)],
                             s_out.at[h])

        def wait_t0(h):
            pltpu.make_async_copy(out_hbm.at[pl.ds(0, C)], acc.at[h],
                                  s_t0.at[h]).wait()

        def wait_adds(h):
            for _ in range(NT - 1):
                pltpu.make_async_copy(out_hbm.at[pl.ds(0, C)], acc.at[h],
                                      s_add.at[h]).wait()

        def wait_out(h):
            pltpu.make_async_copy(acc.at[h], out_hbm.at[pl.ds(0, C)],
                                  s_out.at[h]).wait()

        def pair(i, carry):
            a, b = 2 * i, 2 * i + 1
            sp = jnp.bitwise_and(i, 1)
            sq = 1 - sp
            wait_t0(0)
            issue_adds(sp, 0)

            @pl.when(i > 0)
            def _():
                wait_out(1)

            issue_t0(sp, 1)
            wait_adds(0)
            issue_out(a, 0)
            wait_t0(1)
            issue_adds(sp, 1)

            @pl.when(i + 1 < P)
            def _():
                wait_idx()
                wait_out(0)
                issue_t0(sq, 0)

            wait_adds(1)
            issue_out(b, 1)

            @pl.when(i + 2 < P)
            def _():
                fetch_idx(i + 2, sp)

            return carry

        fetch_idx(0, 0)
        wait_idx()
        issue_t0(0, 0)
        fetch_idx(1, 1)
        lax.fori_loop(0, P, pair, 0)
        wait_out(0)
        wait_out(1)

    return k(idx_packed, *tables)


R = 2048


def _tc_layernorm(hm2, esum, gamma, beta):
    def body(hm_ref, e_ref, g_ref, b_ref, o_ref):
        x = hm_ref[...] + e_ref[...]
        mu = jnp.mean(x, axis=1, keepdims=True)
        xc = x - mu
        var = jnp.mean(xc * xc, axis=1, keepdims=True)
        inv = lax.rsqrt(var + 1e-5)
        o_ref[...] = xc * inv * g_ref[...] + b_ref[...]

    return pl.pallas_call(
        body,
        grid=(N // R,),
        in_specs=[
            pl.BlockSpec((R, D), lambda i: (i, 0)),
            pl.BlockSpec((R, D), lambda i: (i, 0)),
            pl.BlockSpec((1, D), lambda i: (0, 0)),
            pl.BlockSpec((1, D), lambda i: (0, 0)),
        ],
        out_specs=pl.BlockSpec((R, D), lambda i: (i, 0)),
        out_shape=jax.ShapeDtypeStruct((N, D), jnp.float32),
    )(hm2, esum, gamma, beta)


def kernel(target, hm, dhi, dni, ws, rh, t, r_table, dhi_table, dni_table,
           ws_table, rh_table, t_table, gamma, beta):
    i32 = lambda a: a.astype(jnp.int32)
    idx_packed = _tc_pack_indices(i32(target), i32(dhi), i32(dni), i32(ws),
                                  i32(rh), i32(t)).reshape(NT, N // 128, 128)
    tables = _tc_build_tables(r_table, t_table, rh_table, ws_table,
                              dhi_table, dni_table)
    esum = _sc_embed_sum(idx_packed, tables)
    out = _tc_layernorm(
        hm.reshape(N, D), esum,
        gamma.reshape(1, D), beta.reshape(1, D))
    return out.reshape(B, L, D)


# speedup vs baseline: 2.7216x; 1.0775x over previous
import functools

import jax
import jax.numpy as jnp
from jax import lax
from jax.experimental import pallas as pl
from jax.experimental.pallas import tpu as pltpu
from jax.experimental.pallas import tpu_sc as plsc

B, L, D = 4096, 200, 64
N = B * L
NC, NS = 2, 16
NW = NC * NS
W = N // NW
C = 512
KJ = C // 128
P = W // (2 * C)
NT = 3
PVS = [1320, 2280, 2040]

IDXP_B = NT * 2 * KJ * 128 * 4
T0_B = KJ * 128 * D * 4
ADD_B = (NT - 1) * T0_B
OUT_B = C * D * 4


def _tc_pack_indices(target, dhi, dni, ws, rh, t):
    def body(tg_ref, dh_ref, dn_ref, ws_ref, rh_ref, tt_ref, o_ref):
        w = pl.program_id(0)
        o_ref[0] = tg_ref[...] * 13 + tt_ref[...] + w * PVS[0]
        o_ref[1] = rh_ref[...] * 24 + ws_ref[...] + w * PVS[1]
        o_ref[2] = dh_ref[...] * 37 + dn_ref[...] + w * PVS[2]

    bs = pl.BlockSpec((B // NW, L), lambda i: (i, 0))
    return pl.pallas_call(
        body,
        grid=(NW,),
        in_specs=[bs] * 6,
        out_specs=pl.BlockSpec((NT, B // NW, L), lambda i: (0, i, 0)),
        out_shape=jax.ShapeDtypeStruct((NT, B, L), jnp.int32),
    )(target, dhi, dni, ws, rh, t)


def _one_hot_rows(n_rows, va, vb):
    r = lax.broadcasted_iota(jnp.int32, (n_rows, 1), 0)
    oa = (r // vb == lax.broadcasted_iota(jnp.int32, (n_rows, va), 1))
    ob = (r % vb == lax.broadcasted_iota(jnp.int32, (n_rows, vb), 1))
    return oa.astype(jnp.float32), ob.astype(jnp.float32)


def _tc_build_tables(r_table, t_table, rh_table, ws_table, dhi_table,
                     dni_table):
    def body(ra_ref, tt_ref, rh_ref, ws_ref, dh_ref, dn_ref,
             oa_ref, ob_ref, oc_ref):
        def pair(a_ref, b_ref, n_rows, va, vb):
            oa, ob = _one_hot_rows(n_rows, va, vb)
            hi = lax.Precision.HIGHEST
            return (jnp.dot(oa, a_ref[...], precision=hi,
                            preferred_element_type=jnp.float32)
                    + jnp.dot(ob, b_ref[...], precision=hi,
                              preferred_element_type=jnp.float32))

        oa_ref[...] = pair(ra_ref, tt_ref, PVS[0], 101, 13)
        ob_ref[...] = pair(rh_ref, ws_ref, PVS[1], 95, 24)
        oc_ref[...] = pair(dh_ref, dn_ref, PVS[2], 55, 37)

    full = lambda s: pl.BlockSpec(s, lambda i: (0, 0))
    return pl.pallas_call(
        body,
        grid=(NW,),
        in_specs=[full((101, D)), full((13, D)), full((95, D)),
                  full((24, D)), full((55, D)), full((37, D))],
        out_specs=[pl.BlockSpec((PVS[0], D), lambda i: (i, 0)),
                   pl.BlockSpec((PVS[1], D), lambda i: (i, 0)),
                   pl.BlockSpec((PVS[2], D), lambda i: (i, 0))],
        out_shape=[jax.ShapeDtypeStruct((NW * PVS[0], D), jnp.float32),
                   jax.ShapeDtypeStruct((NW * PVS[1], D), jnp.float32),
                   jax.ShapeDtypeStruct((NW * PVS[2], D), jnp.float32)],
    )(r_table, t_table, rh_table, ws_table, dhi_table, dni_table)


def _sc_embed_sum(idx_packed, tables):
    mesh = plsc.VectorSubcoreMesh(core_axis_name="c", subcore_axis_name="s")

    @functools.partial(
        pl.kernel,
        out_type=jax.ShapeDtypeStruct((N, D), jnp.float32),
        mesh=mesh,
        scratch_types=[
            pltpu.VMEM((2, NT, 2 * KJ, 128), jnp.int32),
            pltpu.VMEM((2, C, D), jnp.float32),
            pltpu.SemaphoreType.DMA,
            pltpu.SemaphoreType.DMA((2,)),
            pltpu.SemaphoreType.DMA((2,)),
            pltpu.SemaphoreType.DMA((2,)),
        ],
        compiler_params=pltpu.CompilerParams(use_tc_tiling_on_sc=False),
    )
    def k(idx_hbm, t0, t1, t2, out_hbm, idx_v, acc, s_idx, s_t0, s_add,
          s_out):
        tabs = [t0, t1, t2]
        wid = lax.axis_index("s") * NC + lax.axis_index("c")

        def fetch_idx(pair_i, slot):
            r0 = pl.multiple_of(wid * (W // 128) + pair_i * 2 * KJ, 8)
            pltpu.async_copy(idx_hbm.at[:, pl.ds(r0, 2 * KJ), :],
                             idx_v.at[slot], s_idx)

        def wait_idx():
            pltpu.make_async_copy(idx_hbm.at[:, pl.ds(0, 2 * KJ), :],
                                  idx_v.at[0], s_idx).wait()

        def issue_t0(slot, h):
            for j in range(KJ):
                pltpu.async_copy(tabs[0].at[idx_v.at[slot, 0, h * KJ + j]],
                                 acc.at[h, pl.ds(j * 128, 128)], s_t0.at[h])

        def issue_adds(slot, h):
            for t in range(1, NT):
                for j in range(KJ):
                    pltpu.async_copy(
                        tabs[t].at[idx_v.at[slot, t, h * KJ + j]],
                        acc.at[h, pl.ds(j * 128, 128)], s_add.at[h],
                        add=True)

        def issue_out(g, h):
            row0 = wid * W + g * C
            pltpu.async_copy(acc.at[h], out_hbm.at[pl.ds(row0, C)],
                             s_out.at[h])

        def wait_t0(h):
            pltpu.make_async_copy(out_hbm.at[pl.ds(0, C)], acc.at[h],
                                  s_t0.at[h]).wait()

        def wait_adds(h):
            for _ in range(NT - 1):
                pltpu.make_async_copy(out_hbm.at[pl.ds(0, C)], acc.at[h],
                                      s_add.at[h]).wait()

        def wait_out(h):
            pltpu.make_async_copy(acc.at[h], out_hbm.at[pl.ds(0, C)],
                                  s_out.at[h]).wait()

        def pair(i, carry):
            a, b = 2 * i, 2 * i + 1
            sp = jnp.bitwise_and(i, 1)
            sq = 1 - sp
            wait_t0(0)
            issue_adds(sp, 0)

            @pl.when(i > 0)
            def _():
                wait_out(1)

            issue_t0(sp, 1)
            wait_adds(0)
            issue_out(a, 0)
            wait_t0(1)
            issue_adds(sp, 1)

            @pl.when(i + 1 < P)
            def _():
                wait_idx()
                wait_out(0)
                issue_t0(sq, 0)

            wait_adds(1)
            issue_out(b, 1)

            @pl.when(i + 2 < P)
            def _():
                fetch_idx(i + 2, sp)

            return carry

        fetch_idx(0, 0)
        wait_idx()
        issue_t0(0, 0)
        fetch_idx(1, 1)
        lax.fori_loop(0, P, pair, 0)
        wait_out(0)
        wait_out(1)

    return k(idx_packed, *tables)


R = 8192


def _tc_layernorm(hm2, esum, gamma, beta):
    def body(hm_ref, e_ref, g_ref, b_ref, o_ref):
        x = hm_ref[...] + e_ref[...]
        mu = jnp.mean(x, axis=1, keepdims=True)
        xc = x - mu
        var = jnp.mean(xc * xc, axis=1, keepdims=True)
        inv = lax.rsqrt(var + 1e-5)
        o_ref[...] = xc * inv * g_ref[...] + b_ref[...]

    return pl.pallas_call(
        body,
        grid=(N // R,),
        in_specs=[
            pl.BlockSpec((R, D), lambda i: (i, 0)),
            pl.BlockSpec((R, D), lambda i: (i, 0)),
            pl.BlockSpec((1, D), lambda i: (0, 0)),
            pl.BlockSpec((1, D), lambda i: (0, 0)),
        ],
        out_specs=pl.BlockSpec((R, D), lambda i: (i, 0)),
        out_shape=jax.ShapeDtypeStruct((N, D), jnp.float32),
    )(hm2, esum, gamma, beta)


def kernel(target, hm, dhi, dni, ws, rh, t, r_table, dhi_table, dni_table,
           ws_table, rh_table, t_table, gamma, beta):
    i32 = lambda a: a.astype(jnp.int32)
    idx_packed = _tc_pack_indices(i32(target), i32(dhi), i32(dni), i32(ws),
                                  i32(rh), i32(t)).reshape(NT, N // 128, 128)
    tables = _tc_build_tables(r_table, t_table, rh_table, ws_table,
                              dhi_table, dni_table)
    esum = _sc_embed_sum(idx_packed, tables)
    out = _tc_layernorm(hm.reshape(N, D), esum, gamma.reshape(1, D),
                        beta.reshape(1, D))
    return out.reshape(B, L, D)
